# baseline (speedup 1.0000x reference)
import functools

import jax
import jax.numpy as jnp
from jax import lax
from jax.experimental import pallas as pl
from jax.experimental.pallas import tpu as pltpu
from jax.experimental.pallas import tpu_sc as plsc

_NC = 2
_NS = 16
_NW = _NC * _NS
_NB = 2


def _pack_kernel(d_model, vocab, d_pad, cols):
    grid = (vocab + cols - 1) // cols

    def body(wt_ref, out_ref):
        t = jnp.transpose(wt_ref[...], (1, 0))
        out_ref[...] = jnp.concatenate(
            [t, jnp.zeros((cols, d_pad - d_model), jnp.float32)], axis=1)

    return pl.pallas_call(
        body,
        grid=(grid,),
        in_specs=[pl.BlockSpec((d_model, cols), lambda i: (0, i))],
        out_specs=pl.BlockSpec((cols, d_pad), lambda i: (i, 0)),
        out_shape=jax.ShapeDtypeStruct((vocab, d_pad), jnp.float32),
    )


def _gather_kernel(n_total, d_model, d_pad, chunk):
    b_per_w = n_total // _NW
    n_chunks = b_per_w // chunk
    mesh = plsc.VectorSubcoreMesh(core_axis_name="c", subcore_axis_name="s")

    @functools.partial(
        pl.kernel,
        mesh=mesh,
        out_type=jax.ShapeDtypeStruct((n_total, d_pad), jnp.float32),
        scratch_types=[
            pltpu.VMEM((b_per_w,), jnp.int32),
            pltpu.VMEM((_NB, chunk, d_model), jnp.float32),
            pltpu.SemaphoreType.DMA,
            pltpu.SemaphoreType.DMA,
        ],
        compiler_params=pltpu.CompilerParams(use_tc_tiling_on_sc=False),
    )
    def k(idx_hbm, table_hbm, out_hbm, idx_v, rows_v, gsem, ssem):
        wid = lax.axis_index("s") * _NC + lax.axis_index("c")
        base = wid * b_per_w

        def gather_args(i, b):
            return (table_hbm.at[idx_v.at[pl.ds(i * chunk, chunk)]],
                    rows_v.at[b], gsem)

        def store_args(i, b):
            return (rows_v.at[b],
                    out_hbm.at[pl.ds(base + i * chunk, chunk),
                               pl.ds(0, d_model)], ssem)

        def gather(i, b):
            pltpu.async_copy(*gather_args(i, b))

        def gather_wait(i, b):
            pltpu.make_async_copy(*gather_args(i, b)).wait()

        def store(i, b):
            pltpu.async_copy(*store_args(i, b))

        def store_wait(i, b):
            pltpu.make_async_copy(*store_args(i, b)).wait()

        pltpu.sync_copy(idx_hbm.at[pl.ds(base, b_per_w)], idx_v)
        for b in range(_NB):
            gather(b, b)

        @pl.loop(0, n_chunks - _NB, step=_NB)
        def _(i0):
            for b in range(_NB):
                i = i0 + b
                gather_wait(i, b)
                store(i, b)
                store_wait(i, b)
                gather(i + _NB, b)

        for b in range(_NB):
            i = n_chunks - _NB + b
            gather_wait(i, b)
            store(i, b)
        for b in range(_NB):
            store_wait(n_chunks - _NB + b, b)

    return k


def kernel(ids, emb_weight):
    batch, hist = ids.shape
    vocab, d_model = emb_weight.shape
    d_pad = 128
    n_total = batch * hist
    rows_per_pad = d_pad // d_model
    idx = ids.reshape(n_total).astype(jnp.int32) * rows_per_pad
    table_p = _pack_kernel(d_model, vocab, d_pad, 32768)(emb_weight.T)
    table_v = table_p.reshape(table_p.shape[0] * rows_per_pad, d_model)
    out = _gather_kernel(n_total, d_model, d_pad, 800)(idx, table_v)
    return out[:, :d_model].reshape(batch, hist, d_model)


# speedup vs baseline: 1.0018x; 1.0018x over previous
import functools

import jax
import jax.numpy as jnp
from jax import lax
from jax.experimental import pallas as pl
from jax.experimental.pallas import tpu as pltpu
from jax.experimental.pallas import tpu_sc as plsc

_NC = 2
_NS = 16
_NW = _NC * _NS
_NB = 2


def _pack_kernel(d_model, vocab, d_pad, cols):
    grid = (vocab + cols - 1) // cols

    def body(wt_ref, out_ref):
        t = jnp.transpose(wt_ref[...], (1, 0))
        out_ref[...] = jnp.concatenate(
            [t, jnp.zeros((cols, d_pad - d_model), jnp.float32)], axis=1)

    return pl.pallas_call(
        body,
        grid=(grid,),
        in_specs=[pl.BlockSpec((d_model, cols), lambda i: (0, i))],
        out_specs=pl.BlockSpec((cols, d_pad), lambda i: (i, 0)),
        out_shape=jax.ShapeDtypeStruct((vocab, d_pad), jnp.float32),
    )


def _gather_kernel(n_total, d_model, d_pad, chunk):
    b_per_w = n_total // _NW
    n_chunks = b_per_w // chunk
    mesh = plsc.VectorSubcoreMesh(core_axis_name="c", subcore_axis_name="s")

    @functools.partial(
        pl.kernel,
        mesh=mesh,
        out_type=jax.ShapeDtypeStruct((n_total, d_pad), jnp.float32),
        scratch_types=[
            pltpu.VMEM((b_per_w,), jnp.int32),
            pltpu.VMEM((_NB, chunk, d_model), jnp.float32),
            pltpu.SemaphoreType.DMA,
            pltpu.SemaphoreType.DMA,
        ],
        compiler_params=pltpu.CompilerParams(use_tc_tiling_on_sc=False),
    )
    def k(idx_hbm, table_hbm, out_hbm, idx_v, rows_v, gsem, ssem):
        wid = lax.axis_index("s") * _NC + lax.axis_index("c")
        base = wid * b_per_w

        def gather_args(i, b):
            return (table_hbm.at[idx_v.at[pl.ds(i * chunk, chunk)]],
                    rows_v.at[b], gsem)

        def store_args(i, b):
            return (rows_v.at[b],
                    out_hbm.at[pl.ds(base + i * chunk, chunk),
                               pl.ds(0, d_model)], ssem)

        def gather(i, b):
            pltpu.async_copy(*gather_args(i, b))

        def gather_wait(i, b):
            pltpu.make_async_copy(*gather_args(i, b)).wait()

        def store(i, b):
            pltpu.async_copy(*store_args(i, b))

        def store_wait(i, b):
            pltpu.make_async_copy(*store_args(i, b)).wait()

        pltpu.sync_copy(idx_hbm.at[pl.ds(base, b_per_w)], idx_v)
        for b in range(_NB):
            gather(b, b)

        @pl.loop(0, n_chunks - _NB, step=_NB)
        def _(i0):
            for b in range(_NB):
                i = i0 + b
                gather_wait(i, b)
                store(i, b)
                store_wait(i, b)
                gather(i + _NB, b)

        for b in range(_NB):
            i = n_chunks - _NB + b
            gather_wait(i, b)
            store(i, b)
        for b in range(_NB):
            store_wait(n_chunks - _NB + b, b)

    return k


def kernel(ids, emb_weight):
    batch, hist = ids.shape
    vocab, d_model = emb_weight.shape
    d_pad = 128
    n_total = batch * hist
    rows_per_pad = d_pad // d_model
    idx = ids.reshape(n_total).astype(jnp.int32) * rows_per_pad
    table_p = _pack_kernel(d_model, vocab, d_pad, 32768)(emb_weight.T)
    table_v = table_p.reshape(table_p.shape[0] * rows_per_pad, d_model)
    out = _gather_kernel(n_total, d_model, d_pad, 512)(idx, table_v)
    return out[:, :d_model].reshape(batch, hist, d_model)


# speedup vs baseline: 1.0453x; 1.0435x over previous
import functools

import jax
import jax.numpy as jnp
from jax import lax
from jax.experimental import pallas as pl
from jax.experimental.pallas import tpu as pltpu
from jax.experimental.pallas import tpu_sc as plsc

_NC = 2
_NS = 16
_NW = _NC * _NS
_NB = 2


def _pack_kernel(d_model, vocab, d_pad, cols):
    grid = (vocab + cols - 1) // cols
    half = cols // 2

    def body(wt_ref, out_ref):
        t = jnp.transpose(wt_ref[...], (1, 0))
        out_ref[...] = jnp.concatenate([t[:half], t[half:]], axis=1)

    return pl.pallas_call(
        body,
        grid=(grid,),
        in_specs=[pl.BlockSpec((d_model, cols), lambda i: (0, i))],
        out_specs=pl.BlockSpec((half, d_pad), lambda i: (i, 0)),
        out_shape=jax.ShapeDtypeStruct((grid * half, d_pad), jnp.float32),
    )


def _gather_kernel(n_total, d_model, d_pad, chunk):
    b_per_w = n_total // _NW
    n_chunks = b_per_w // chunk
    mesh = plsc.VectorSubcoreMesh(core_axis_name="c", subcore_axis_name="s")

    @functools.partial(
        pl.kernel,
        mesh=mesh,
        out_type=jax.ShapeDtypeStruct((n_total, d_pad), jnp.float32),
        scratch_types=[
            pltpu.VMEM((b_per_w,), jnp.int32),
            pltpu.VMEM((_NB, chunk, d_model), jnp.float32),
            pltpu.SemaphoreType.DMA,
            pltpu.SemaphoreType.DMA,
        ],
        compiler_params=pltpu.CompilerParams(use_tc_tiling_on_sc=False),
    )
    def k(idx_hbm, table_hbm, out_hbm, idx_v, rows_v, gsem, ssem):
        wid = lax.axis_index("s") * _NC + lax.axis_index("c")
        base = wid * b_per_w

        def gather_args(i, b):
            return (table_hbm.at[idx_v.at[pl.ds(i * chunk, chunk)]],
                    rows_v.at[b], gsem)

        def store_args(i, b):
            return (rows_v.at[b],
                    out_hbm.at[pl.ds(base + i * chunk, chunk),
                               pl.ds(0, d_model)], ssem)

        def gather(i, b):
            pltpu.async_copy(*gather_args(i, b))

        def gather_wait(i, b):
            pltpu.make_async_copy(*gather_args(i, b)).wait()

        def store(i, b):
            pltpu.async_copy(*store_args(i, b))

        def store_wait(i, b):
            pltpu.make_async_copy(*store_args(i, b)).wait()

        pltpu.sync_copy(idx_hbm.at[pl.ds(base, b_per_w)], idx_v)
        for b in range(_NB):
            gather(b, b)

        @pl.loop(0, n_chunks - _NB, step=_NB)
        def _(i0):
            for b in range(_NB):
                i = i0 + b
                gather_wait(i, b)
                store(i, b)
                store_wait(i, b)
                gather(i + _NB, b)

        for b in range(_NB):
            i = n_chunks - _NB + b
            gather_wait(i, b)
            store(i, b)
        for b in range(_NB):
            store_wait(n_chunks - _NB + b, b)

    return k


def kernel(ids, emb_weight):
    batch, hist = ids.shape
    vocab, d_model = emb_weight.shape
    d_pad = 128
    n_total = batch * hist
    cols = 32768
    half = cols // 2
    i = ids.reshape(n_total).astype(jnp.int32)
    idx = (i // cols) * cols + (i % half) * 2 + (i // half) % 2
    table_p = _pack_kernel(d_model, vocab, d_pad, cols)(emb_weight.T)
    table_v = table_p.reshape(table_p.shape[0] * 2, d_model)
    out = _gather_kernel(n_total, d_model, d_pad, 512)(idx, table_v)
    return out[:, :d_model].reshape(batch, hist, d_model)
